# baseline (speedup 1.0000x reference)
import functools

import jax
import jax.numpy as jnp
from jax import lax
from jax.experimental import pallas as pl
from jax.experimental.pallas import tpu as pltpu

_BM = 200


def _body(adj_ref, f_ref, wb_ref, wcat_ref, noise_ref, o_ref, s_ref,
          *, nb, d_emb):
    i = pl.program_id(0)
    d2 = 2 * d_emb

    @pl.when(i == 0)
    def _phase_p():
        wc = jnp.dot(wb_ref[...], wcat_ref[...],
                     preferred_element_type=jnp.float32)
        s_ref[:, :d2] = jnp.dot(f_ref[...], wc,
                                preferred_element_type=jnp.float32)

    @pl.when((i >= 1) & (i <= nb))
    def _phase_g():
        r = (i - 1) * _BM
        s_ref[pl.ds(r, _BM), d2:2 * d2] = jnp.dot(
            adj_ref[...], s_ref[:, :d2],
            preferred_element_type=jnp.float32)

    @pl.when((i >= nb + 1) & (i <= 2 * nb))
    def _phase_z():
        r = (i - 1 - nb) * _BM
        m = jnp.maximum(jnp.dot(adj_ref[...], s_ref[:, d2:2 * d2],
                                preferred_element_type=jnp.float32), 0.0)
        mean = m[:, :d_emb]
        logstd = m[:, d_emb:]
        s_ref[pl.ds(r, _BM), 2 * d2:2 * d2 + d_emb] = (
            noise_ref[...] * jnp.exp(logstd) + mean)

    @pl.when(i > 2 * nb)
    def _phase_out():
        r = (i - 1 - 2 * nb) * _BM
        zi = s_ref[pl.ds(r, _BM), 2 * d2:2 * d2 + d_emb]
        zall = s_ref[:, 2 * d2:2 * d2 + d_emb]
        o_ref[...] = lax.dot_general(
            zi, zall, (((1,), (1,)), ((), ())),
            preferred_element_type=jnp.float32)


def kernel(adj, features, W_base, W_mean, W_logstd, noise):
    n, d_in = features.shape
    d_hid = W_base.shape[1]
    d_emb = W_mean.shape[1]
    d2 = 2 * d_emb
    nb = n // _BM

    w_cat = jnp.concatenate([W_mean, W_logstd], axis=1)

    def adj_map(i):
        return (jnp.where(i <= nb, jnp.maximum(i - 1, 0),
                          jnp.where(i <= 2 * nb, i - 1 - nb, nb - 1)), 0)

    def noise_map(i):
        return (jnp.clip(i - 1 - nb, 0, nb - 1), 0)

    def out_map(i):
        return (jnp.where(i > 2 * nb, i - 1 - 2 * nb, 0), 0)

    body = functools.partial(_body, nb=nb, d_emb=d_emb)

    out = pl.pallas_call(
        body,
        grid=(3 * nb + 1,),
        in_specs=[
            pl.BlockSpec((_BM, n), adj_map),
            pl.BlockSpec((n, d_in), lambda i: (0, 0)),
            pl.BlockSpec((d_in, d_hid), lambda i: (0, 0)),
            pl.BlockSpec((d_hid, d2), lambda i: (0, 0)),
            pl.BlockSpec((_BM, d_emb), noise_map),
        ],
        out_specs=pl.BlockSpec((_BM, n), out_map),
        out_shape=jax.ShapeDtypeStruct((n, n), jnp.float32),
        scratch_shapes=[
            pltpu.VMEM((n, 2 * d2 + d_emb), jnp.float32),
        ],
    )(adj, features, W_base, w_cat, noise)

    return out


# speedup vs baseline: 1.0153x; 1.0153x over previous
import functools

import jax
import jax.numpy as jnp
from jax import lax
from jax.experimental import pallas as pl
from jax.experimental.pallas import tpu as pltpu

_BM = 200


def _body(adj_ref, f_ref, wb_ref, wcat_ref, noise_ref, o_ref,
          p_ref, g_ref, z_ref, *, nb, d_emb):
    i = pl.program_id(0)

    @pl.when(i == 0)
    def _phase_p():
        wc = jnp.dot(wb_ref[...], wcat_ref[...],
                     preferred_element_type=jnp.float32)
        p_ref[...] = jnp.dot(f_ref[...], wc,
                             preferred_element_type=jnp.float32)

    @pl.when((i >= 1) & (i <= nb))
    def _phase_g():
        r = (i - 1) * _BM
        g_ref[pl.ds(r, _BM), :] = jnp.dot(
            adj_ref[...], p_ref[...],
            preferred_element_type=jnp.float32)

    @pl.when((i >= nb + 1) & (i <= 2 * nb))
    def _phase_z():
        r = (i - 1 - nb) * _BM
        m = jnp.maximum(jnp.dot(adj_ref[...], g_ref[...],
                                preferred_element_type=jnp.float32), 0.0)
        mean = m[:, :d_emb]
        logstd = m[:, d_emb:]
        z_ref[pl.ds(r, _BM), :] = (
            noise_ref[...] * jnp.exp(logstd) + mean)

    @pl.when(i > 2 * nb)
    def _phase_out():
        r = (i - 1 - 2 * nb) * _BM
        zi = z_ref[pl.ds(r, _BM), :]
        o_ref[...] = lax.dot_general(
            zi, z_ref[...], (((1,), (1,)), ((), ())),
            preferred_element_type=jnp.float32)


def kernel(adj, features, W_base, W_mean, W_logstd, noise):
    n, d_in = features.shape
    d_hid = W_base.shape[1]
    d_emb = W_mean.shape[1]
    d2 = 2 * d_emb
    nb = n // _BM

    w_cat = jnp.concatenate([W_mean, W_logstd], axis=1)

    def adj_map(i):
        return (jnp.where(i <= nb, jnp.maximum(i - 1, 0),
                          jnp.where(i <= 2 * nb, i - 1 - nb, nb - 1)), 0)

    def noise_map(i):
        return (jnp.clip(i - 1 - nb, 0, nb - 1), 0)

    def out_map(i):
        return (jnp.where(i > 2 * nb, i - 1 - 2 * nb, 0), 0)

    body = functools.partial(_body, nb=nb, d_emb=d_emb)

    out = pl.pallas_call(
        body,
        grid=(3 * nb + 1,),
        in_specs=[
            pl.BlockSpec((_BM, n), adj_map),
            pl.BlockSpec((n, d_in), lambda i: (0, 0)),
            pl.BlockSpec((d_in, d_hid), lambda i: (0, 0)),
            pl.BlockSpec((d_hid, d2), lambda i: (0, 0)),
            pl.BlockSpec((_BM, d_emb), noise_map),
        ],
        out_specs=pl.BlockSpec((_BM, n), out_map),
        out_shape=jax.ShapeDtypeStruct((n, n), jnp.float32),
        scratch_shapes=[
            pltpu.VMEM((n, d2), jnp.float32),
            pltpu.VMEM((n, d2), jnp.float32),
            pltpu.VMEM((n, d_emb), jnp.float32),
        ],
    )(adj, features, W_base, w_cat, noise)

    return out
